# baseline (speedup 1.0000x reference)
import functools

import jax
import jax.numpy as jnp
from jax import lax
from jax.experimental import pallas as pl
from jax.experimental.pallas import tpu as pltpu
from jax.experimental.pallas import tpu_sc as plsc

NC = 2
NS = 16
NW = NC * NS
CH = 128


def _sc_degree(dstp, zeros1d):
    E2 = dstp.shape[0]
    N = zeros1d.shape[0]
    per_w = E2 // NW
    n_vec = per_w // 16

    mesh = plsc.VectorSubcoreMesh(
        core_axis_name="c", subcore_axis_name="s", num_cores=NC, num_subcores=NS)

    @functools.partial(
        pl.kernel,
        out_type=jax.ShapeDtypeStruct((NW * N,), jnp.float32),
        mesh=mesh,
        compiler_params=pltpu.CompilerParams(needs_layout_passes=False),
        scratch_types=[
            pltpu.VMEM((per_w,), jnp.int32),
            pltpu.VMEM((N + 128,), jnp.float32),
        ],
    )
    def k(dst_hbm, z_hbm, out_hbm, idx_v, hist_v):
        cid = lax.axis_index("c")
        sid = lax.axis_index("s")
        wid = cid * NS + sid
        pltpu.sync_copy(z_hbm, hist_v.at[pl.ds(0, N)])
        for j in range(8):
            hist_v[pl.ds(N + j * 16, 16)] = jnp.zeros((16,), jnp.float32)
        pltpu.sync_copy(dst_hbm.at[pl.ds(wid * per_w, per_w)], idx_v)
        ones = jnp.ones((16,), jnp.float32)

        def body(j, carry):
            iv = idx_v[pl.ds(j * 16, 16)]
            plsc.addupdate_scatter(hist_v, [iv], ones)
            return carry

        lax.fori_loop(0, n_vec, body, 0)
        pltpu.sync_copy(hist_v.at[pl.ds(0, N)], out_hbm.at[pl.ds(wid * N, N)])

    return k(dstp, zeros1d)


def _sc_edge_agg(ms, src1, dst1, zerosN):
    N, H = ms.shape
    E2 = src1.shape[0]
    per_w = E2 // NW
    n_ch = per_w // CH
    rps = (N // NS) // 8 * 8
    rtail = N - rps * NS

    mesh = plsc.VectorSubcoreMesh(
        core_axis_name="c", subcore_axis_name="s", num_cores=NC, num_subcores=NS)

    @functools.partial(
        pl.kernel,
        out_type=jax.ShapeDtypeStruct((NC, N, H), jnp.float32),
        mesh=mesh,
        scratch_types=[
            pltpu.VMEM((CH,), jnp.int32),
            pltpu.VMEM((CH,), jnp.int32),
            pltpu.VMEM((CH, H), jnp.float32),
            pltpu.VMEM_SHARED((N + 128, H), jnp.float32),
            pltpu.SemaphoreType.DMA,
        ],
    )
    def k(ms_hbm, src_hbm, dst_hbm, z_hbm, out_hbm,
          sidx, didx, rows, acc_sh, sem):
        cid = lax.axis_index("c")
        sid = lax.axis_index("s")
        base = (cid * NS + sid) * per_w
        pltpu.sync_copy(z_hbm.at[pl.ds(sid * rps, rps)],
                        acc_sh.at[pl.ds(sid * rps, rps)])
        if rtail:
            @pl.when(sid == 0)
            def _():
                pltpu.sync_copy(z_hbm.at[pl.ds(rps * NS, rtail)],
                                acc_sh.at[pl.ds(rps * NS, rtail)])
        plsc.subcore_barrier()

        def body(t, carry):
            off = base + t * CH
            pltpu.sync_copy(src_hbm.at[pl.ds(off, CH)], sidx)
            pltpu.sync_copy(dst_hbm.at[pl.ds(off, CH)], didx)
            pltpu.async_copy(ms_hbm.at[sidx], rows, sem).wait()
            pltpu.sync_copy(rows, acc_sh.at[didx], add=True)
            return carry

        lax.fori_loop(0, n_ch, body, 0)
        plsc.subcore_barrier()
        pltpu.sync_copy(acc_sh.at[pl.ds(sid * rps, rps)],
                        out_hbm.at[cid, pl.ds(sid * rps, rps)])
        if rtail:
            @pl.when(sid == 0)
            def _():
                pltpu.sync_copy(acc_sh.at[pl.ds(rps * NS, rtail)],
                                out_hbm.at[cid, pl.ds(rps * NS, rtail)])

    return k(ms, src1, dst1, zerosN)


def _sc_ctx_gather(fv, cidx):
    N, H = fv.shape
    T = cidx.shape[0]
    full = T // CH
    rem = T - full * CH
    n_w = full // NW
    extra = full % NW

    mesh = plsc.VectorSubcoreMesh(
        core_axis_name="c", subcore_axis_name="s", num_cores=NC, num_subcores=NS)

    @functools.partial(
        pl.kernel,
        out_type=jax.ShapeDtypeStruct((T, H), jnp.float32),
        mesh=mesh,
        scratch_types=[
            pltpu.VMEM((CH,), jnp.int32),
            pltpu.VMEM((rem if rem else 8,), jnp.int32),
            pltpu.VMEM((CH, H), jnp.float32),
            pltpu.SemaphoreType.DMA,
        ],
    )
    def k(fv_hbm, cidx_hbm, out_hbm, idx_v, idx_r, rows, sem):
        cid = lax.axis_index("c")
        sid = lax.axis_index("s")
        wid = cid * NS + sid

        def body(t, carry):
            off = (t * NW + wid) * CH
            pltpu.sync_copy(cidx_hbm.at[pl.ds(off, CH)], idx_v)
            pltpu.async_copy(fv_hbm.at[idx_v], rows, sem).wait()
            pltpu.sync_copy(rows, out_hbm.at[pl.ds(off, CH)])
            return carry

        nt = n_w + jnp.where(wid < extra, 1, 0).astype(jnp.int32)
        lax.fori_loop(0, nt, body, 0)
        if rem:
            @pl.when(wid == NW - 1)
            def _():
                off = full * CH
                pltpu.sync_copy(cidx_hbm.at[pl.ds(off, rem)], idx_r)
                pltpu.async_copy(fv_hbm.at[idx_r], rows.at[pl.ds(0, rem)], sem).wait()
                pltpu.sync_copy(rows.at[pl.ds(0, rem)], out_hbm.at[pl.ds(off, rem)])

    return k(fv, cidx)


def _dinv_from_degp(degp_blk):
    deg = jnp.sum(degp_blk, axis=0)[0, 0] + 1.0
    return lax.rsqrt(deg)


def _tc_scale_matmul(x, W, degp, blk=1000):
    N, H = x.shape

    def body(x_ref, w_ref, degp_ref, out_ref):
        dinv = _dinv_from_degp(degp_ref[...])
        m = jnp.dot(x_ref[...], w_ref[...], preferred_element_type=jnp.float32)
        out_ref[...] = m * dinv[:, None]

    return pl.pallas_call(
        body,
        grid=(N // blk,),
        in_specs=[
            pl.BlockSpec((blk, H), lambda i: (i, 0)),
            pl.BlockSpec((H, H), lambda i: (0, 0)),
            pl.BlockSpec((NW, 1, 1, blk), lambda i: (0, i, 0, 0)),
        ],
        out_specs=pl.BlockSpec((blk, H), lambda i: (i, 0)),
        out_shape=jax.ShapeDtypeStruct((N, H), jnp.float32),
    )(x, W, degp)


def _tc_combine_relu_matmul(aggp, ms, degp, W, b2, blk=1000):
    N, H = ms.shape

    def body(aggp_ref, ms_ref, degp_ref, w_ref, b_ref, out_ref):
        dinv = _dinv_from_degp(degp_ref[...])
        agg = aggp_ref[0] + aggp_ref[1] + ms_ref[...]
        h1 = jnp.maximum(agg * dinv[:, None] + b_ref[...], 0.0)
        m = jnp.dot(h1, w_ref[...], preferred_element_type=jnp.float32)
        out_ref[...] = m * dinv[:, None]

    return pl.pallas_call(
        body,
        grid=(N // blk,),
        in_specs=[
            pl.BlockSpec((NC, blk, H), lambda i: (0, i, 0)),
            pl.BlockSpec((blk, H), lambda i: (i, 0)),
            pl.BlockSpec((NW, 1, 1, blk), lambda i: (0, i, 0, 0)),
            pl.BlockSpec((H, H), lambda i: (0, 0)),
            pl.BlockSpec((1, H), lambda i: (0, 0)),
        ],
        out_specs=pl.BlockSpec((blk, H), lambda i: (i, 0)),
        out_shape=jax.ShapeDtypeStruct((N, H), jnp.float32),
    )(aggp, ms, degp, W, b2)


def _tc_combine_final(aggp, ms2, degp, b2, blk=1000):
    N, H = ms2.shape

    def body(aggp_ref, ms_ref, degp_ref, b_ref, out_ref):
        dinv = _dinv_from_degp(degp_ref[...])
        agg = aggp_ref[0] + aggp_ref[1] + ms_ref[...]
        out_ref[...] = agg * dinv[:, None] + b_ref[...]

    return pl.pallas_call(
        body,
        grid=(N // blk,),
        in_specs=[
            pl.BlockSpec((NC, blk, H), lambda i: (0, i, 0)),
            pl.BlockSpec((blk, H), lambda i: (i, 0)),
            pl.BlockSpec((NW, 1, 1, blk), lambda i: (0, i, 0, 0)),
            pl.BlockSpec((1, H), lambda i: (0, 0)),
        ],
        out_specs=pl.BlockSpec((blk, H), lambda i: (i, 0)),
        out_shape=jax.ShapeDtypeStruct((N, H), jnp.float32),
    )(aggp, ms2, degp, b2)


def _tc_resize_mul(ctx2d, Wr, br2, fv, blk=400):
    N, K = ctx2d.shape
    H = Wr.shape[1]

    def body(ctx_ref, wr_ref, br_ref, fv_ref, out_ref):
        r = jnp.dot(ctx_ref[...], wr_ref[...], preferred_element_type=jnp.float32)
        out_ref[...] = fv_ref[...] * (r + br_ref[...])

    return pl.pallas_call(
        body,
        grid=(N // blk,),
        in_specs=[
            pl.BlockSpec((blk, K), lambda i: (i, 0)),
            pl.BlockSpec((K, H), lambda i: (0, 0)),
            pl.BlockSpec((1, H), lambda i: (0, 0)),
            pl.BlockSpec((blk, H), lambda i: (i, 0)),
        ],
        out_specs=pl.BlockSpec((blk, H), lambda i: (i, 0)),
        out_shape=jax.ShapeDtypeStruct((N, H), jnp.float32),
    )(ctx2d, Wr, br2, fv)


def kernel(x, W, b, Wr, br, edge_index, context_idx):
    N, H = x.shape
    MC = context_idx.shape[1]
    src = edge_index[0]
    dst = edge_index[1]
    zeros1d = jnp.zeros((N,), jnp.float32)
    zerosN = jnp.zeros((N, H), jnp.float32)
    b2 = b.reshape(1, H)
    br2 = br.reshape(1, H)
    blk = 1000

    E = src.shape[0]
    assert E % NW == 0
    per_w = E // NW
    n_ch = -(-per_w // CH)
    n_ch = (n_ch + 3) // 4 * 4
    pad_w = n_ch * CH - per_w
    srcp = jnp.concatenate(
        [src.reshape(NW, per_w), jnp.zeros((NW, pad_w), src.dtype)], axis=1)
    padv = (N + jnp.arange(pad_w, dtype=dst.dtype) % 128)[None, :]
    dstp = jnp.concatenate(
        [dst.reshape(NW, per_w), jnp.broadcast_to(padv, (NW, pad_w))], axis=1)
    src1 = srcp.reshape(-1)
    dst1 = dstp.reshape(-1)

    degp1d = _sc_degree(dstp.reshape(-1), zeros1d)
    degp = degp1d.reshape(NW, N // blk, 1, blk)
    ms = _tc_scale_matmul(x, W, degp)
    aggp1 = _sc_edge_agg(ms, src1, dst1, zerosN)
    ms2 = _tc_combine_relu_matmul(aggp1, ms, degp, W, b2)
    aggp2 = _sc_edge_agg(ms2, src1, dst1, zerosN)
    fv = _tc_combine_final(aggp2, ms2, degp, b2)
    ctx = _sc_ctx_gather(fv, context_idx.reshape(-1))
    rep = _tc_resize_mul(ctx.reshape(N, MC * H), Wr, br2, fv)
    return rep


# speedup vs baseline: 2.6596x; 2.6596x over previous
import functools

import jax
import jax.numpy as jnp
from jax import lax
from jax.experimental import pallas as pl
from jax.experimental.pallas import tpu as pltpu
from jax.experimental.pallas import tpu_sc as plsc

NC = 2
NS = 16
NW = NC * NS
CH = 128


def _sc_degree(dstp, zeros1d):
    E2 = dstp.shape[0]
    N = zeros1d.shape[0]
    per_w = E2 // NW
    n_vec = per_w // 16

    mesh = plsc.VectorSubcoreMesh(
        core_axis_name="c", subcore_axis_name="s", num_cores=NC, num_subcores=NS)

    @functools.partial(
        pl.kernel,
        out_type=jax.ShapeDtypeStruct((NW * N,), jnp.float32),
        mesh=mesh,
        compiler_params=pltpu.CompilerParams(needs_layout_passes=False),
        scratch_types=[
            pltpu.VMEM((per_w,), jnp.int32),
            pltpu.VMEM((N + 128,), jnp.float32),
        ],
    )
    def k(dst_hbm, z_hbm, out_hbm, idx_v, hist_v):
        cid = lax.axis_index("c")
        sid = lax.axis_index("s")
        wid = cid * NS + sid
        pltpu.sync_copy(z_hbm, hist_v.at[pl.ds(0, N)])
        for j in range(8):
            hist_v[pl.ds(N + j * 16, 16)] = jnp.zeros((16,), jnp.float32)
        pltpu.sync_copy(dst_hbm.at[pl.ds(wid * per_w, per_w)], idx_v)
        ones = jnp.ones((16,), jnp.float32)

        def body(j, carry):
            iv = idx_v[pl.ds(j * 16, 16)]
            plsc.addupdate_scatter(hist_v, [iv], ones)
            return carry

        lax.fori_loop(0, n_vec, body, 0)
        pltpu.sync_copy(hist_v.at[pl.ds(0, N)], out_hbm.at[pl.ds(wid * N, N)])

    return k(dstp, zeros1d)


def _sc_edge_agg(ms, src1, dst1, zerosN):
    N, H = ms.shape
    E2 = src1.shape[0]
    per_w = E2 // NW
    n_ch = per_w // CH
    rem = per_w - n_ch * CH
    assert n_ch % 2 == 0 and n_ch >= 4
    rps = (N // NS) // 8 * 8
    rtail = N - rps * NS

    mesh = plsc.VectorSubcoreMesh(
        core_axis_name="c", subcore_axis_name="s", num_cores=NC, num_subcores=NS)

    @functools.partial(
        pl.kernel,
        out_type=jax.ShapeDtypeStruct((NC, N, H), jnp.float32),
        mesh=mesh,
        scratch_types=[
            pltpu.VMEM((CH,), jnp.int32),
            pltpu.VMEM((CH,), jnp.int32),
            pltpu.VMEM((CH,), jnp.int32),
            pltpu.VMEM((CH,), jnp.int32),
            pltpu.VMEM((rem if rem else 8,), jnp.int32),
            pltpu.VMEM((rem if rem else 8,), jnp.int32),
            pltpu.VMEM((CH, H), jnp.float32),
            pltpu.VMEM((CH, H), jnp.float32),
            pltpu.VMEM_SHARED((N, H), jnp.float32),
            pltpu.SemaphoreType.DMA,
            pltpu.SemaphoreType.DMA,
            pltpu.SemaphoreType.DMA,
            pltpu.SemaphoreType.DMA,
        ],
    )
    def k(ms_hbm, src_hbm, dst_hbm, z_hbm, out_hbm,
          sidx0, didx0, sidx1, didx1, sidx_r, didx_r, rows0, rows1, acc_sh,
          gsem0, gsem1, isem0, isem1):
        cid = lax.axis_index("c")
        sid = lax.axis_index("s")
        base = (cid * NS + sid) * per_w
        pltpu.sync_copy(z_hbm.at[pl.ds(sid * rps, rps)],
                        acc_sh.at[pl.ds(sid * rps, rps)])
        if rtail:
            @pl.when(sid == 0)
            def _():
                pltpu.sync_copy(z_hbm.at[pl.ds(rps * NS, rtail)],
                                acc_sh.at[pl.ds(rps * NS, rtail)])
        plsc.subcore_barrier()

        pltpu.sync_copy(src_hbm.at[pl.ds(base, CH)], sidx0)
        pltpu.sync_copy(dst_hbm.at[pl.ds(base, CH)], didx0)
        pltpu.async_copy(ms_hbm.at[sidx0], rows0, gsem0)
        pltpu.async_copy(src_hbm.at[pl.ds(base + CH, CH)], sidx1, isem1)
        pltpu.async_copy(dst_hbm.at[pl.ds(base + CH, CH)], didx1, isem1)

        def body(g, carry):
            t0 = 2 * g
            t1 = t0 + 1
            o1 = base + t1 * CH
            pltpu.make_async_copy(src_hbm.at[pl.ds(o1, CH)], sidx1, isem1).wait()
            pltpu.make_async_copy(dst_hbm.at[pl.ds(o1, CH)], didx1, isem1).wait()
            pltpu.async_copy(ms_hbm.at[sidx1], rows1, gsem1)
            pltpu.make_async_copy(ms_hbm.at[sidx0], rows0, gsem0).wait()
            pltpu.sync_copy(rows0, acc_sh.at[didx0], add=True)
            o2 = base + jnp.minimum(t0 + 2, n_ch - 2) * CH
            pltpu.async_copy(src_hbm.at[pl.ds(o2, CH)], sidx0, isem0)
            pltpu.async_copy(dst_hbm.at[pl.ds(o2, CH)], didx0, isem0)
            pltpu.make_async_copy(src_hbm.at[pl.ds(o2, CH)], sidx0, isem0).wait()
            pltpu.make_async_copy(dst_hbm.at[pl.ds(o2, CH)], didx0, isem0).wait()

            @pl.when(t1 + 1 < n_ch)
            def _():
                pltpu.async_copy(ms_hbm.at[sidx0], rows0, gsem0)
            pltpu.make_async_copy(ms_hbm.at[sidx1], rows1, gsem1).wait()
            pltpu.sync_copy(rows1, acc_sh.at[didx1], add=True)
            o3 = base + jnp.minimum(t1 + 2, n_ch - 1) * CH
            pltpu.async_copy(src_hbm.at[pl.ds(o3, CH)], sidx1, isem1)
            pltpu.async_copy(dst_hbm.at[pl.ds(o3, CH)], didx1, isem1)
            return carry

        lax.fori_loop(0, n_ch // 2, body, 0)
        o_last = base + (n_ch - 1) * CH
        pltpu.make_async_copy(src_hbm.at[pl.ds(o_last, CH)], sidx1, isem1).wait()
        pltpu.make_async_copy(dst_hbm.at[pl.ds(o_last, CH)], didx1, isem1).wait()
        if rem:
            off = base + n_ch * CH
            pltpu.sync_copy(src_hbm.at[pl.ds(off, rem)], sidx_r)
            pltpu.sync_copy(dst_hbm.at[pl.ds(off, rem)], didx_r)
            pltpu.async_copy(
                ms_hbm.at[sidx_r], rows0.at[pl.ds(0, rem)], gsem0).wait()
            pltpu.sync_copy(rows0.at[pl.ds(0, rem)], acc_sh.at[didx_r], add=True)
        plsc.subcore_barrier()
        pltpu.sync_copy(acc_sh.at[pl.ds(sid * rps, rps)],
                        out_hbm.at[cid, pl.ds(sid * rps, rps)])
        if rtail:
            @pl.when(sid == 0)
            def _():
                pltpu.sync_copy(acc_sh.at[pl.ds(rps * NS, rtail)],
                                out_hbm.at[cid, pl.ds(rps * NS, rtail)])

    return k(ms, src1, dst1, zerosN)


def _sc_ctx_gather(fv, cidx):
    N, H = fv.shape
    T = cidx.shape[0]
    full = T // CH
    rem = T - full * CH
    n_w = full // NW
    extra = full % NW

    mesh = plsc.VectorSubcoreMesh(
        core_axis_name="c", subcore_axis_name="s", num_cores=NC, num_subcores=NS)

    @functools.partial(
        pl.kernel,
        out_type=jax.ShapeDtypeStruct((T, H), jnp.float32),
        mesh=mesh,
        scratch_types=[
            pltpu.VMEM((CH,), jnp.int32),
            pltpu.VMEM((rem if rem else 8,), jnp.int32),
            pltpu.VMEM((CH, H), jnp.float32),
            pltpu.SemaphoreType.DMA,
        ],
    )
    def k(fv_hbm, cidx_hbm, out_hbm, idx_v, idx_r, rows, sem):
        cid = lax.axis_index("c")
        sid = lax.axis_index("s")
        wid = cid * NS + sid

        def body(t, carry):
            off = (t * NW + wid) * CH
            pltpu.sync_copy(cidx_hbm.at[pl.ds(off, CH)], idx_v)
            pltpu.async_copy(fv_hbm.at[idx_v], rows, sem).wait()
            pltpu.sync_copy(rows, out_hbm.at[pl.ds(off, CH)])
            return carry

        nt = n_w + jnp.where(wid < extra, 1, 0).astype(jnp.int32)
        lax.fori_loop(0, nt, body, 0)
        if rem:
            @pl.when(wid == NW - 1)
            def _():
                off = full * CH
                pltpu.sync_copy(cidx_hbm.at[pl.ds(off, rem)], idx_r)
                pltpu.async_copy(fv_hbm.at[idx_r], rows.at[pl.ds(0, rem)], sem).wait()
                pltpu.sync_copy(rows.at[pl.ds(0, rem)], out_hbm.at[pl.ds(off, rem)])

    return k(fv, cidx)


def _dinv_from_degp(degp_blk):
    deg = jnp.sum(degp_blk, axis=0)[0, 0] + 1.0
    return lax.rsqrt(deg)


def _tc_scale_matmul(x, W, degp, blk=1000):
    N, H = x.shape

    def body(x_ref, w_ref, degp_ref, out_ref):
        dinv = _dinv_from_degp(degp_ref[...])
        m = jnp.dot(x_ref[...], w_ref[...], preferred_element_type=jnp.float32)
        out_ref[...] = m * dinv[:, None]

    return pl.pallas_call(
        body,
        grid=(N // blk,),
        in_specs=[
            pl.BlockSpec((blk, H), lambda i: (i, 0)),
            pl.BlockSpec((H, H), lambda i: (0, 0)),
            pl.BlockSpec((NW, 1, 1, blk), lambda i: (0, i, 0, 0)),
        ],
        out_specs=pl.BlockSpec((blk, H), lambda i: (i, 0)),
        out_shape=jax.ShapeDtypeStruct((N, H), jnp.float32),
    )(x, W, degp)


def _tc_combine_relu_matmul(aggp, ms, degp, W, b2, blk=1000):
    N, H = ms.shape

    def body(aggp_ref, ms_ref, degp_ref, w_ref, b_ref, out_ref):
        dinv = _dinv_from_degp(degp_ref[...])
        agg = aggp_ref[0] + aggp_ref[1] + ms_ref[...]
        h1 = jnp.maximum(agg * dinv[:, None] + b_ref[...], 0.0)
        m = jnp.dot(h1, w_ref[...], preferred_element_type=jnp.float32)
        out_ref[...] = m * dinv[:, None]

    return pl.pallas_call(
        body,
        grid=(N // blk,),
        in_specs=[
            pl.BlockSpec((NC, blk, H), lambda i: (0, i, 0)),
            pl.BlockSpec((blk, H), lambda i: (i, 0)),
            pl.BlockSpec((NW, 1, 1, blk), lambda i: (0, i, 0, 0)),
            pl.BlockSpec((H, H), lambda i: (0, 0)),
            pl.BlockSpec((1, H), lambda i: (0, 0)),
        ],
        out_specs=pl.BlockSpec((blk, H), lambda i: (i, 0)),
        out_shape=jax.ShapeDtypeStruct((N, H), jnp.float32),
    )(aggp, ms, degp, W, b2)


def _tc_combine_final(aggp, ms2, degp, b2, blk=1000):
    N, H = ms2.shape

    def body(aggp_ref, ms_ref, degp_ref, b_ref, out_ref):
        dinv = _dinv_from_degp(degp_ref[...])
        agg = aggp_ref[0] + aggp_ref[1] + ms_ref[...]
        out_ref[...] = agg * dinv[:, None] + b_ref[...]

    return pl.pallas_call(
        body,
        grid=(N // blk,),
        in_specs=[
            pl.BlockSpec((NC, blk, H), lambda i: (0, i, 0)),
            pl.BlockSpec((blk, H), lambda i: (i, 0)),
            pl.BlockSpec((NW, 1, 1, blk), lambda i: (0, i, 0, 0)),
            pl.BlockSpec((1, H), lambda i: (0, 0)),
        ],
        out_specs=pl.BlockSpec((blk, H), lambda i: (i, 0)),
        out_shape=jax.ShapeDtypeStruct((N, H), jnp.float32),
    )(aggp, ms2, degp, b2)


def _tc_resize_mul(ctx2d, Wr, br2, fv, blk=400):
    N, K = ctx2d.shape
    H = Wr.shape[1]

    def body(ctx_ref, wr_ref, br_ref, fv_ref, out_ref):
        r = jnp.dot(ctx_ref[...], wr_ref[...], preferred_element_type=jnp.float32)
        out_ref[...] = fv_ref[...] * (r + br_ref[...])

    return pl.pallas_call(
        body,
        grid=(N // blk,),
        in_specs=[
            pl.BlockSpec((blk, K), lambda i: (i, 0)),
            pl.BlockSpec((K, H), lambda i: (0, 0)),
            pl.BlockSpec((1, H), lambda i: (0, 0)),
            pl.BlockSpec((blk, H), lambda i: (i, 0)),
        ],
        out_specs=pl.BlockSpec((blk, H), lambda i: (i, 0)),
        out_shape=jax.ShapeDtypeStruct((N, H), jnp.float32),
    )(ctx2d, Wr, br2, fv)


def kernel(x, W, b, Wr, br, edge_index, context_idx):
    N, H = x.shape
    MC = context_idx.shape[1]
    src = edge_index[0]
    dst = edge_index[1]
    zeros1d = jnp.zeros((N,), jnp.float32)
    zerosN = jnp.zeros((N, H), jnp.float32)
    b2 = b.reshape(1, H)
    br2 = br.reshape(1, H)
    blk = 1000

    E = src.shape[0]
    assert E % NW == 0

    degp1d = _sc_degree(dst, zeros1d)
    degp = degp1d.reshape(NW, N // blk, 1, blk)
    ms = _tc_scale_matmul(x, W, degp)
    aggp1 = _sc_edge_agg(ms, src, dst, zerosN)
    ms2 = _tc_combine_relu_matmul(aggp1, ms, degp, W, b2)
    aggp2 = _sc_edge_agg(ms2, src, dst, zerosN)
    fv = _tc_combine_final(aggp2, ms2, degp, b2)
    ctx = _sc_ctx_gather(fv, context_idx.reshape(-1))
    rep = _tc_resize_mul(ctx.reshape(N, MC * H), Wr, br2, fv)
    return rep


# speedup vs baseline: 2.7876x; 1.0481x over previous
import functools

import jax
import jax.numpy as jnp
from jax import lax
from jax.experimental import pallas as pl
from jax.experimental.pallas import tpu as pltpu
from jax.experimental.pallas import tpu_sc as plsc

NC = 2
NS = 16
NW = NC * NS
CH = 128


def _sc_degree(dstp, zeros1d):
    E2 = dstp.shape[0]
    N = zeros1d.shape[0]
    per_w = E2 // NW
    n_vec = per_w // 16

    mesh = plsc.VectorSubcoreMesh(
        core_axis_name="c", subcore_axis_name="s", num_cores=NC, num_subcores=NS)

    @functools.partial(
        pl.kernel,
        out_type=jax.ShapeDtypeStruct((NW * N,), jnp.float32),
        mesh=mesh,
        compiler_params=pltpu.CompilerParams(needs_layout_passes=False),
        scratch_types=[
            pltpu.VMEM((per_w,), jnp.int32),
            pltpu.VMEM((N + 128,), jnp.float32),
        ],
    )
    def k(dst_hbm, z_hbm, out_hbm, idx_v, hist_v):
        cid = lax.axis_index("c")
        sid = lax.axis_index("s")
        wid = cid * NS + sid
        pltpu.sync_copy(z_hbm, hist_v.at[pl.ds(0, N)])
        for j in range(8):
            hist_v[pl.ds(N + j * 16, 16)] = jnp.zeros((16,), jnp.float32)
        pltpu.sync_copy(dst_hbm.at[pl.ds(wid * per_w, per_w)], idx_v)
        ones = jnp.ones((16,), jnp.float32)

        def body(j, carry):
            iv = idx_v[pl.ds(j * 16, 16)]
            plsc.addupdate_scatter(hist_v, [iv], ones)
            return carry

        lax.fori_loop(0, n_vec, body, 0)
        pltpu.sync_copy(hist_v.at[pl.ds(0, N)], out_hbm.at[pl.ds(wid * N, N)])

    return k(dstp, zeros1d)


def _sc_edge_agg(ms, src1, dst1, zerosN):
    N, H = ms.shape
    E2 = src1.shape[0]
    per_w = E2 // NW
    n_ch = per_w // CH
    rem = per_w - n_ch * CH
    assert n_ch % 2 == 0 and n_ch >= 4
    rps = (N // NS) // 8 * 8
    rtail = N - rps * NS

    mesh = plsc.VectorSubcoreMesh(
        core_axis_name="c", subcore_axis_name="s", num_cores=NC, num_subcores=NS)

    @functools.partial(
        pl.kernel,
        out_type=jax.ShapeDtypeStruct((NC, N, H), jnp.float32),
        mesh=mesh,
        scratch_types=[
            pltpu.VMEM((CH,), jnp.int32),
            pltpu.VMEM((CH,), jnp.int32),
            pltpu.VMEM((CH,), jnp.int32),
            pltpu.VMEM((CH,), jnp.int32),
            pltpu.VMEM((rem if rem else 8,), jnp.int32),
            pltpu.VMEM((rem if rem else 8,), jnp.int32),
            pltpu.VMEM((CH, H), jnp.float32),
            pltpu.VMEM((CH, H), jnp.float32),
            pltpu.VMEM_SHARED((N, H), jnp.float32),
            pltpu.SemaphoreType.DMA,
            pltpu.SemaphoreType.DMA,
            pltpu.SemaphoreType.DMA,
            pltpu.SemaphoreType.DMA,
        ],
    )
    def k(ms_hbm, src_hbm, dst_hbm, z_hbm, out_hbm,
          sidx0, didx0, sidx1, didx1, sidx_r, didx_r, rows0, rows1, acc_sh,
          gsem0, gsem1, isem0, isem1):
        cid = lax.axis_index("c")
        sid = lax.axis_index("s")
        base = (cid * NS + sid) * per_w
        pltpu.sync_copy(z_hbm.at[pl.ds(sid * rps, rps)],
                        acc_sh.at[pl.ds(sid * rps, rps)])
        if rtail:
            @pl.when(sid == 0)
            def _():
                pltpu.sync_copy(z_hbm.at[pl.ds(rps * NS, rtail)],
                                acc_sh.at[pl.ds(rps * NS, rtail)])
        plsc.subcore_barrier()

        pltpu.sync_copy(src_hbm.at[pl.ds(base, CH)], sidx0)
        pltpu.sync_copy(dst_hbm.at[pl.ds(base, CH)], didx0)
        pltpu.async_copy(ms_hbm.at[sidx0], rows0, gsem0)
        pltpu.async_copy(src_hbm.at[pl.ds(base + CH, CH)], sidx1, isem1)
        pltpu.async_copy(dst_hbm.at[pl.ds(base + CH, CH)], didx1, isem1)

        def body(g, carry):
            t0 = 2 * g
            t1 = t0 + 1
            o1 = base + t1 * CH
            pltpu.make_async_copy(src_hbm.at[pl.ds(o1, CH)], sidx1, isem1).wait()
            pltpu.make_async_copy(dst_hbm.at[pl.ds(o1, CH)], didx1, isem1).wait()
            pltpu.async_copy(ms_hbm.at[sidx1], rows1, gsem1)
            pltpu.make_async_copy(ms_hbm.at[sidx0], rows0, gsem0).wait()
            pltpu.sync_copy(rows0, acc_sh.at[didx0], add=True)
            o2 = base + jnp.minimum(t0 + 2, n_ch - 2) * CH
            pltpu.async_copy(src_hbm.at[pl.ds(o2, CH)], sidx0, isem0)
            pltpu.async_copy(dst_hbm.at[pl.ds(o2, CH)], didx0, isem0)
            pltpu.make_async_copy(src_hbm.at[pl.ds(o2, CH)], sidx0, isem0).wait()
            pltpu.make_async_copy(dst_hbm.at[pl.ds(o2, CH)], didx0, isem0).wait()

            @pl.when(t1 + 1 < n_ch)
            def _():
                pltpu.async_copy(ms_hbm.at[sidx0], rows0, gsem0)
            pltpu.make_async_copy(ms_hbm.at[sidx1], rows1, gsem1).wait()
            pltpu.sync_copy(rows1, acc_sh.at[didx1], add=True)
            o3 = base + jnp.minimum(t1 + 2, n_ch - 1) * CH
            pltpu.async_copy(src_hbm.at[pl.ds(o3, CH)], sidx1, isem1)
            pltpu.async_copy(dst_hbm.at[pl.ds(o3, CH)], didx1, isem1)
            return carry

        lax.fori_loop(0, n_ch // 2, body, 0)
        o_last = base + (n_ch - 1) * CH
        pltpu.make_async_copy(src_hbm.at[pl.ds(o_last, CH)], sidx1, isem1).wait()
        pltpu.make_async_copy(dst_hbm.at[pl.ds(o_last, CH)], didx1, isem1).wait()
        if rem:
            off = base + n_ch * CH
            pltpu.sync_copy(src_hbm.at[pl.ds(off, rem)], sidx_r)
            pltpu.sync_copy(dst_hbm.at[pl.ds(off, rem)], didx_r)
            pltpu.async_copy(
                ms_hbm.at[sidx_r], rows0.at[pl.ds(0, rem)], gsem0).wait()
            pltpu.sync_copy(rows0.at[pl.ds(0, rem)], acc_sh.at[didx_r], add=True)
        plsc.subcore_barrier()
        pltpu.sync_copy(acc_sh.at[pl.ds(sid * rps, rps)],
                        out_hbm.at[cid, pl.ds(sid * rps, rps)])
        if rtail:
            @pl.when(sid == 0)
            def _():
                pltpu.sync_copy(acc_sh.at[pl.ds(rps * NS, rtail)],
                                out_hbm.at[cid, pl.ds(rps * NS, rtail)])

    return k(ms, src1, dst1, zerosN)


def _sc_ctx_gather(fv, cidx):
    N, H = fv.shape
    T = cidx.shape[0]
    full = T // CH
    rem = T - full * CH
    n_w = full // NW
    extra = full % NW
    assert n_w >= 2

    mesh = plsc.VectorSubcoreMesh(
        core_axis_name="c", subcore_axis_name="s", num_cores=NC, num_subcores=NS)

    @functools.partial(
        pl.kernel,
        out_type=jax.ShapeDtypeStruct((T, H), jnp.float32),
        mesh=mesh,
        scratch_types=[
            pltpu.VMEM((CH,), jnp.int32),
            pltpu.VMEM((CH,), jnp.int32),
            pltpu.VMEM((rem if rem else 8,), jnp.int32),
            pltpu.VMEM((CH, H), jnp.float32),
            pltpu.VMEM((CH, H), jnp.float32),
            pltpu.SemaphoreType.DMA,
            pltpu.SemaphoreType.DMA,
            pltpu.SemaphoreType.DMA,
            pltpu.SemaphoreType.DMA,
        ],
    )
    def k(fv_hbm, cidx_hbm, out_hbm, idx0, idx1, idx_r, rows0, rows1,
          gsem0, gsem1, isem0, isem1):
        cid = lax.axis_index("c")
        sid = lax.axis_index("s")
        wid = cid * NS + sid
        nt = n_w + jnp.where(wid < extra, 1, 0).astype(jnp.int32)

        def off(t):
            return (t * NW + wid) * CH

        pltpu.sync_copy(cidx_hbm.at[pl.ds(off(0), CH)], idx0)
        pltpu.async_copy(fv_hbm.at[idx0], rows0, gsem0)
        pltpu.async_copy(cidx_hbm.at[pl.ds(off(1), CH)], idx1, isem1)

        def body(g, carry):
            t0 = 2 * g
            t1 = t0 + 1
            pltpu.make_async_copy(
                cidx_hbm.at[pl.ds(off(t1), CH)], idx1, isem1).wait()
            pltpu.async_copy(fv_hbm.at[idx1], rows1, gsem1)
            pltpu.make_async_copy(fv_hbm.at[idx0], rows0, gsem0).wait()
            pltpu.sync_copy(rows0, out_hbm.at[pl.ds(off(t0), CH)])
            o2 = off(jnp.minimum(t0 + 2, nt - 1))
            pltpu.async_copy(cidx_hbm.at[pl.ds(o2, CH)], idx0, isem0)
            pltpu.make_async_copy(cidx_hbm.at[pl.ds(o2, CH)], idx0, isem0).wait()

            @pl.when(t1 + 1 < 2 * (nt // 2))
            def _():
                pltpu.async_copy(fv_hbm.at[idx0], rows0, gsem0)
            pltpu.make_async_copy(fv_hbm.at[idx1], rows1, gsem1).wait()
            pltpu.sync_copy(rows1, out_hbm.at[pl.ds(off(t1), CH)])
            o3 = off(jnp.minimum(t1 + 2, nt - 1))
            pltpu.async_copy(cidx_hbm.at[pl.ds(o3, CH)], idx1, isem1)
            return carry

        lax.fori_loop(0, nt // 2, body, 0)
        pltpu.make_async_copy(
            cidx_hbm.at[pl.ds(off(nt - 1), CH)], idx1, isem1).wait()

        @pl.when(nt % 2 == 1)
        def _():
            t = nt - 1
            pltpu.sync_copy(cidx_hbm.at[pl.ds(off(t), CH)], idx0)
            pltpu.async_copy(fv_hbm.at[idx0], rows0, gsem0).wait()
            pltpu.sync_copy(rows0, out_hbm.at[pl.ds(off(t), CH)])

        if rem:
            @pl.when(wid == NW - 1)
            def _():
                o = full * CH
                pltpu.sync_copy(cidx_hbm.at[pl.ds(o, rem)], idx_r)
                pltpu.async_copy(
                    fv_hbm.at[idx_r], rows0.at[pl.ds(0, rem)], gsem0).wait()
                pltpu.sync_copy(rows0.at[pl.ds(0, rem)],
                                out_hbm.at[pl.ds(o, rem)])

    return k(fv, cidx)


def _dinv_from_degp(degp_blk):
    deg = jnp.sum(degp_blk, axis=0)[0, 0] + 1.0
    return lax.rsqrt(deg)


def _tc_scale_matmul(x, W, degp, blk=1000):
    N, H = x.shape

    def body(x_ref, w_ref, degp_ref, out_ref):
        dinv = _dinv_from_degp(degp_ref[...])
        m = jnp.dot(x_ref[...], w_ref[...], preferred_element_type=jnp.float32)
        out_ref[...] = m * dinv[:, None]

    return pl.pallas_call(
        body,
        grid=(N // blk,),
        in_specs=[
            pl.BlockSpec((blk, H), lambda i: (i, 0)),
            pl.BlockSpec((H, H), lambda i: (0, 0)),
            pl.BlockSpec((NW, 1, 1, blk), lambda i: (0, i, 0, 0)),
        ],
        out_specs=pl.BlockSpec((blk, H), lambda i: (i, 0)),
        out_shape=jax.ShapeDtypeStruct((N, H), jnp.float32),
    )(x, W, degp)


def _tc_combine_relu_matmul(aggp, ms, degp, W, b2, blk=1000):
    N, H = ms.shape

    def body(aggp_ref, ms_ref, degp_ref, w_ref, b_ref, out_ref):
        dinv = _dinv_from_degp(degp_ref[...])
        agg = aggp_ref[0] + aggp_ref[1] + ms_ref[...]
        h1 = jnp.maximum(agg * dinv[:, None] + b_ref[...], 0.0)
        m = jnp.dot(h1, w_ref[...], preferred_element_type=jnp.float32)
        out_ref[...] = m * dinv[:, None]

    return pl.pallas_call(
        body,
        grid=(N // blk,),
        in_specs=[
            pl.BlockSpec((NC, blk, H), lambda i: (0, i, 0)),
            pl.BlockSpec((blk, H), lambda i: (i, 0)),
            pl.BlockSpec((NW, 1, 1, blk), lambda i: (0, i, 0, 0)),
            pl.BlockSpec((H, H), lambda i: (0, 0)),
            pl.BlockSpec((1, H), lambda i: (0, 0)),
        ],
        out_specs=pl.BlockSpec((blk, H), lambda i: (i, 0)),
        out_shape=jax.ShapeDtypeStruct((N, H), jnp.float32),
    )(aggp, ms, degp, W, b2)


def _tc_combine_final(aggp, ms2, degp, b2, blk=1000):
    N, H = ms2.shape

    def body(aggp_ref, ms_ref, degp_ref, b_ref, out_ref):
        dinv = _dinv_from_degp(degp_ref[...])
        agg = aggp_ref[0] + aggp_ref[1] + ms_ref[...]
        out_ref[...] = agg * dinv[:, None] + b_ref[...]

    return pl.pallas_call(
        body,
        grid=(N // blk,),
        in_specs=[
            pl.BlockSpec((NC, blk, H), lambda i: (0, i, 0)),
            pl.BlockSpec((blk, H), lambda i: (i, 0)),
            pl.BlockSpec((NW, 1, 1, blk), lambda i: (0, i, 0, 0)),
            pl.BlockSpec((1, H), lambda i: (0, 0)),
        ],
        out_specs=pl.BlockSpec((blk, H), lambda i: (i, 0)),
        out_shape=jax.ShapeDtypeStruct((N, H), jnp.float32),
    )(aggp, ms2, degp, b2)


def _tc_resize_mul(ctx2d, Wr, br2, fv, blk=400):
    N, K = ctx2d.shape
    H = Wr.shape[1]

    def body(ctx_ref, wr_ref, br_ref, fv_ref, out_ref):
        r = jnp.dot(ctx_ref[...], wr_ref[...], preferred_element_type=jnp.float32)
        out_ref[...] = fv_ref[...] * (r + br_ref[...])

    return pl.pallas_call(
        body,
        grid=(N // blk,),
        in_specs=[
            pl.BlockSpec((blk, K), lambda i: (i, 0)),
            pl.BlockSpec((K, H), lambda i: (0, 0)),
            pl.BlockSpec((1, H), lambda i: (0, 0)),
            pl.BlockSpec((blk, H), lambda i: (i, 0)),
        ],
        out_specs=pl.BlockSpec((blk, H), lambda i: (i, 0)),
        out_shape=jax.ShapeDtypeStruct((N, H), jnp.float32),
    )(ctx2d, Wr, br2, fv)


def kernel(x, W, b, Wr, br, edge_index, context_idx):
    N, H = x.shape
    MC = context_idx.shape[1]
    src = edge_index[0]
    dst = edge_index[1]
    zeros1d = jnp.zeros((N,), jnp.float32)
    zerosN = jnp.zeros((N, H), jnp.float32)
    b2 = b.reshape(1, H)
    br2 = br.reshape(1, H)
    blk = 1000

    E = src.shape[0]
    assert E % NW == 0

    degp1d = _sc_degree(dst, zeros1d)
    degp = degp1d.reshape(NW, N // blk, 1, blk)
    ms = _tc_scale_matmul(x, W, degp)
    aggp1 = _sc_edge_agg(ms, src, dst, zerosN)
    ms2 = _tc_combine_relu_matmul(aggp1, ms, degp, W, b2)
    aggp2 = _sc_edge_agg(ms2, src, dst, zerosN)
    fv = _tc_combine_final(aggp2, ms2, degp, b2)
    ctx = _sc_ctx_gather(fv, context_idx.reshape(-1))
    rep = _tc_resize_mul(ctx.reshape(N, MC * H), Wr, br2, fv)
    return rep
